# baseline (speedup 1.0000x reference)
import functools

import jax
import jax.numpy as jnp
from jax import lax
from jax.experimental import pallas as pl
from jax.experimental.pallas import tpu as pltpu
from jax.experimental.pallas import tpu_sc as plsc

_B = 4096
_D = 32
_PACK = 128 // _D

_INFO = plsc.get_sparse_core_info()
_NC = _INFO.num_cores
_NS = _INFO.num_subcores
_NW = _NC * _NS
_BPW = _B // _NW
_L = 16


@functools.partial(
    pl.kernel,
    mesh=plsc.VectorSubcoreMesh(core_axis_name="c", subcore_axis_name="s"),
    out_type=[
        jax.ShapeDtypeStruct((_B, _D), jnp.float32),
        jax.ShapeDtypeStruct((_B, _D), jnp.float32),
    ],
    scratch_types=[
        pltpu.VMEM((_BPW,), jnp.int32),
        pltpu.VMEM((_BPW,), jnp.int32),
        pltpu.VMEM((_BPW, 128), jnp.float32),
        pltpu.VMEM((_BPW, _D), jnp.float32),
        pltpu.SemaphoreType.DMA,
    ],
    compiler_params=pltpu.CompilerParams(needs_layout_passes=False),
)
def _gather_latents(user_ids, item_ids, user_table, item_table,
                    u_out, i_out, idx_v, row_v, rows_v, out_v, sem):
    wid = lax.axis_index("s") * _NC + lax.axis_index("c")
    base = wid * _BPW

    def one_table(ids_hbm, table_hbm, out_hbm):
        pltpu.sync_copy(ids_hbm.at[pl.ds(base, _BPW)], idx_v)
        for g in range(_BPW // _L):
            ids16 = idx_v[pl.ds(g * _L, _L)]
            row_v[pl.ds(g * _L, _L)] = lax.shift_right_logical(ids16, 2)
        pltpu.async_copy(table_hbm.at[row_v], rows_v, sem).wait()
        rvec = lax.iota(jnp.int32, _L)
        for g in range(_BPW // _L):
            ids16 = idx_v[pl.ds(g * _L, _L)]
            off = (ids16 & 3) * 32
            rows16 = lax.add(rvec, g * _L)
            for c in range(_D):
                vals = plsc.load_gather(rows_v, [rows16, lax.add(off, c)])
                plsc.store_scatter(out_v, [rows16, lax.full((_L,), c, jnp.int32)],
                                   vals)
        pltpu.sync_copy(out_v, out_hbm.at[pl.ds(base, _BPW)])

    one_table(user_ids, user_table, u_out)
    one_table(item_ids, item_table, i_out)


def _mm_body(u_ref, it_ref, o_ref):
    o_ref[...] = lax.dot_general(
        u_ref[...], it_ref[...], (((1,), (1,)), ((), ())),
        preferred_element_type=jnp.float32)


_BM = 1024
_BN = 1024

_matmul = pl.pallas_call(
    _mm_body,
    grid=(_B // _BM, _B // _BN),
    in_specs=[
        pl.BlockSpec((_BM, _D), lambda i, j: (i, 0)),
        pl.BlockSpec((_BN, _D), lambda i, j: (j, 0)),
    ],
    out_specs=pl.BlockSpec((_BM, _BN), lambda i, j: (i, j)),
    out_shape=jax.ShapeDtypeStruct((_B, _B), jnp.float32),
)


def kernel(user_ids, item_ids, user_table, item_table):
    u128 = user_table.reshape(-1, 128)
    i128 = item_table.reshape(-1, 128)
    u_lat, i_lat = _gather_latents(user_ids, item_ids, u128, i128)
    return _matmul(u_lat, i_lat)


# speedup vs baseline: 5.1333x; 5.1333x over previous
import functools

import jax
import jax.numpy as jnp
from jax import lax
from jax.experimental import pallas as pl
from jax.experimental.pallas import tpu as pltpu
from jax.experimental.pallas import tpu_sc as plsc

_B = 4096
_D = 32

_INFO = plsc.get_sparse_core_info()
_NC = _INFO.num_cores
_NS = _INFO.num_subcores
_NW = _NC * _NS
_BPW = _B // _NW
_G = 16


@functools.partial(
    pl.kernel,
    mesh=plsc.VectorSubcoreMesh(core_axis_name="c", subcore_axis_name="s"),
    out_type=[
        jax.ShapeDtypeStruct((_D, _B), jnp.float32),
        jax.ShapeDtypeStruct((_D, _B), jnp.float32),
    ],
    scratch_types=[
        pltpu.VMEM((_BPW,), jnp.int32),
        pltpu.VMEM((_G, _D, 128), jnp.float32),
        pltpu.VMEM((_D, _BPW), jnp.float32),
        pltpu.SemaphoreType.DMA,
    ],
    compiler_params=pltpu.CompilerParams(needs_layout_passes=False),
)
def _gather_latents(user_ids, item_ids, user_table_t, item_table_t,
                    u_out_t, i_out_t, idx_v, tiles_v, out_v, sem):
    wid = lax.axis_index("s") * _NC + lax.axis_index("c")
    base = wid * _BPW
    iv = lax.iota(jnp.int32, 16)

    def one_table(ids_hbm, tbl_t, out_t_hbm):
        pltpu.sync_copy(ids_hbm.at[pl.ds(base, _BPW)], idx_v)

        def body(g, _):
            v16 = idx_v[pl.ds(g * _G, _G)]
            lanes = v16 & 127
            for j in range(_G):
                start = pl.multiple_of(
                    lax.shift_right_logical(v16[j], 7) * 128, 128)
                pltpu.async_copy(tbl_t.at[:, pl.ds(start, 128)],
                                 tiles_v.at[j], sem)
            for j in range(_G):
                pltpu.make_async_copy(tbl_t.at[:, pl.ds(0, 128)],
                                      tiles_v.at[j], sem).wait()
            for j in range(_G):
                slot16 = jnp.full((16,), j, jnp.int32)
                lane16 = jnp.full((16,), lanes[j], jnp.int32)
                col16 = jnp.full((16,), g * _G + j, jnp.int32)
                for h in range(2):
                    rows16 = iv + h * 16
                    v = plsc.load_gather(tiles_v, [slot16, rows16, lane16])
                    plsc.store_scatter(out_v, [rows16, col16], v)
            return 0

        lax.fori_loop(0, _BPW // _G, body, 0)
        pltpu.sync_copy(out_v, out_t_hbm.at[:, pl.ds(base, _BPW)])

    one_table(user_ids, user_table_t, u_out_t)
    one_table(item_ids, item_table_t, i_out_t)


def _mm_body(u_ref, it_ref, o_ref):
    o_ref[...] = lax.dot_general(
        u_ref[...], it_ref[...], (((0,), (0,)), ((), ())),
        preferred_element_type=jnp.float32)


_BM = 1024
_BN = 1024

_matmul = pl.pallas_call(
    _mm_body,
    grid=(_B // _BM, _B // _BN),
    in_specs=[
        pl.BlockSpec((_D, _BM), lambda i, j: (0, i)),
        pl.BlockSpec((_D, _BN), lambda i, j: (0, j)),
    ],
    out_specs=pl.BlockSpec((_BM, _BN), lambda i, j: (i, j)),
    out_shape=jax.ShapeDtypeStruct((_B, _B), jnp.float32),
)


def kernel(user_ids, item_ids, user_table, item_table):
    u_lat_t, i_lat_t = _gather_latents(user_ids, item_ids,
                                       user_table.T, item_table.T)
    return _matmul(u_lat_t, i_lat_t)


# speedup vs baseline: 5.5951x; 1.0900x over previous
import functools

import jax
import jax.numpy as jnp
from jax import lax
from jax.experimental import pallas as pl
from jax.experimental.pallas import tpu as pltpu
from jax.experimental.pallas import tpu_sc as plsc

_B = 4096
_D = 32

_INFO = plsc.get_sparse_core_info()
_NC = _INFO.num_cores
_NS = _INFO.num_subcores
_NW = _NC * _NS
_BPW = _B // _NW
_G = 16


@functools.partial(
    pl.kernel,
    mesh=plsc.VectorSubcoreMesh(core_axis_name="c", subcore_axis_name="s"),
    out_type=[
        jax.ShapeDtypeStruct((_D, _B), jnp.float32),
        jax.ShapeDtypeStruct((_D, _B), jnp.float32),
    ],
    scratch_types=[
        pltpu.VMEM((_BPW,), jnp.int32),
        pltpu.VMEM((_G, _D, 128), jnp.float32),
        pltpu.VMEM((_D, _BPW), jnp.float32),
        pltpu.SemaphoreType.DMA,
    ],
    compiler_params=pltpu.CompilerParams(needs_layout_passes=False),
)
def _gather_latents(user_ids, item_ids, user_table_t, item_table_t,
                    u_out_t, i_out_t, idx_v, tiles_v, out_v, sem):
    wid = lax.axis_index("s") * _NC + lax.axis_index("c")
    base = wid * _BPW
    iv = lax.iota(jnp.int32, 16)

    def one_table(ids_hbm, tbl_t, out_t_hbm):
        pltpu.sync_copy(ids_hbm.at[pl.ds(base, _BPW)], idx_v)

        def body(g, _):
            v16 = idx_v[pl.ds(g * _G, _G)]
            lanes = v16 & 127
            for j in range(_G):
                start = pl.multiple_of(
                    lax.shift_right_logical(v16[j], 7) * 128, 128)
                pltpu.async_copy(tbl_t.at[:, pl.ds(start, 128)],
                                 tiles_v.at[j], sem)
            for j in range(_G):
                pltpu.make_async_copy(tbl_t.at[:, pl.ds(0, 128)],
                                      tiles_v.at[j], sem).wait()
                slot16 = jnp.full((16,), j, jnp.int32)
                lane16 = jnp.full((16,), lanes[j], jnp.int32)
                col16 = jnp.full((16,), g * _G + j, jnp.int32)
                for h in range(2):
                    rows16 = iv + h * 16
                    v = plsc.load_gather(tiles_v, [slot16, rows16, lane16])
                    plsc.store_scatter(out_v, [rows16, col16], v)
            return 0

        lax.fori_loop(0, _BPW // _G, body, 0)
        pltpu.sync_copy(out_v, out_t_hbm.at[:, pl.ds(base, _BPW)])

    one_table(user_ids, user_table_t, u_out_t)
    one_table(item_ids, item_table_t, i_out_t)


def _mm_body(u_ref, it_ref, o_ref):
    o_ref[...] = lax.dot_general(
        u_ref[...], it_ref[...], (((0,), (0,)), ((), ())),
        preferred_element_type=jnp.float32)


_BM = 1024
_BN = 4096

_matmul = pl.pallas_call(
    _mm_body,
    grid=(_B // _BM,),
    in_specs=[
        pl.BlockSpec((_D, _BM), lambda i: (0, i)),
        pl.BlockSpec((_D, _BN), lambda i: (0, 0)),
    ],
    out_specs=pl.BlockSpec((_BM, _BN), lambda i: (i, 0)),
    out_shape=jax.ShapeDtypeStruct((_B, _B), jnp.float32),
)


def kernel(user_ids, item_ids, user_table, item_table):
    u_lat_t, i_lat_t = _gather_latents(user_ids, item_ids,
                                       user_table.T, item_table.T)
    return _matmul(u_lat_t, i_lat_t)


# speedup vs baseline: 6.0001x; 1.0724x over previous
import functools

import jax
import jax.numpy as jnp
from jax import lax
from jax.experimental import pallas as pl
from jax.experimental.pallas import tpu as pltpu
from jax.experimental.pallas import tpu_sc as plsc

_B = 4096
_D = 32

_INFO = plsc.get_sparse_core_info()
_NC = _INFO.num_cores
_NS = _INFO.num_subcores
_NW = _NC * _NS
_BPW = _B // _NW
_G = 16


@functools.partial(
    pl.kernel,
    mesh=plsc.VectorSubcoreMesh(core_axis_name="c", subcore_axis_name="s"),
    out_type=[
        jax.ShapeDtypeStruct((_D, _B), jnp.float32),
        jax.ShapeDtypeStruct((_D, _B), jnp.float32),
    ],
    scratch_types=[
        pltpu.VMEM((_BPW,), jnp.int32),
        pltpu.VMEM((_G, _D, 128), jnp.float32),
        pltpu.VMEM((_D, _BPW), jnp.float32),
        pltpu.SemaphoreType.DMA,
    ],
    compiler_params=pltpu.CompilerParams(needs_layout_passes=False),
)
def _gather_latents(user_ids, item_ids, user_table_t, item_table_t,
                    u_out_t, i_out_t, idx_v, tiles_v, out_v, sem):
    wid = lax.axis_index("s") * _NC + lax.axis_index("c")
    base = wid * _BPW
    iv = lax.iota(jnp.int32, 16)

    def one_table(ids_hbm, tbl_t, out_t_hbm):
        pltpu.sync_copy(ids_hbm.at[pl.ds(base, _BPW)], idx_v)

        def fire(v16, j):
            start = pl.multiple_of(
                lax.shift_right_logical(v16[j], 7) * 128, 128)
            pltpu.async_copy(tbl_t.at[:, pl.ds(start, 128)],
                             tiles_v.at[j], sem)

        def drain(v16, g, j):
            pltpu.make_async_copy(tbl_t.at[:, pl.ds(0, 128)],
                                  tiles_v.at[j], sem).wait()
            slot16 = jnp.full((16,), j, jnp.int32)
            lane16 = jnp.full((16,), v16[j] & 127, jnp.int32)
            col16 = jnp.full((16,), g * _G + j, jnp.int32)
            for h in range(2):
                rows16 = iv + h * 16
                v = plsc.load_gather(tiles_v, [slot16, rows16, lane16])
                plsc.store_scatter(out_v, [rows16, col16], v)

        v0 = idx_v[pl.ds(0, _G)]
        for j in range(_G):
            fire(v0, j)

        def body(g, v_prev):
            v16 = idx_v[pl.ds(g * _G, _G)]
            for j in range(_G):
                drain(v_prev, g - 1, j)
                fire(v16, j)
            return v16

        vlast = lax.fori_loop(1, _BPW // _G, body, v0)
        for j in range(_G):
            drain(vlast, _BPW // _G - 1, j)
        pltpu.sync_copy(out_v, out_t_hbm.at[:, pl.ds(base, _BPW)])

    one_table(user_ids, user_table_t, u_out_t)
    one_table(item_ids, item_table_t, i_out_t)


def _mm_body(u_ref, it_ref, o_ref):
    o_ref[...] = lax.dot_general(
        u_ref[...], it_ref[...], (((0,), (0,)), ((), ())),
        preferred_element_type=jnp.float32)


_BM = 1024
_BN = 4096

_matmul = pl.pallas_call(
    _mm_body,
    grid=(_B // _BM,),
    in_specs=[
        pl.BlockSpec((_D, _BM), lambda i: (0, i)),
        pl.BlockSpec((_D, _BN), lambda i: (0, 0)),
    ],
    out_specs=pl.BlockSpec((_BM, _BN), lambda i: (i, 0)),
    out_shape=jax.ShapeDtypeStruct((_B, _B), jnp.float32),
)


def kernel(user_ids, item_ids, user_table, item_table):
    u_lat_t, i_lat_t = _gather_latents(user_ids, item_ids,
                                       user_table.T, item_table.T)
    return _matmul(u_lat_t, i_lat_t)


# speedup vs baseline: 6.1044x; 1.0174x over previous
import functools

import jax
import jax.numpy as jnp
from jax import lax
from jax.experimental import pallas as pl
from jax.experimental.pallas import tpu as pltpu
from jax.experimental.pallas import tpu_sc as plsc

_B = 4096
_D = 32

_INFO = plsc.get_sparse_core_info()
_NC = _INFO.num_cores
_NS = _INFO.num_subcores
_NW = _NC * _NS
_BPW = _B // _NW
_G = 16


@functools.partial(
    pl.kernel,
    mesh=plsc.VectorSubcoreMesh(core_axis_name="c", subcore_axis_name="s"),
    out_type=[
        jax.ShapeDtypeStruct((_D, _B), jnp.float32),
        jax.ShapeDtypeStruct((_D, _B), jnp.float32),
    ],
    scratch_types=[
        pltpu.VMEM((2 * _BPW,), jnp.int32),
        pltpu.VMEM((_G, _D, 128), jnp.float32),
        pltpu.VMEM((_D, _BPW), jnp.float32),
        pltpu.VMEM((_D, _BPW), jnp.float32),
        pltpu.SemaphoreType.DMA,
        pltpu.SemaphoreType.DMA,
    ],
    compiler_params=pltpu.CompilerParams(needs_layout_passes=False),
)
def _gather_latents(user_ids, item_ids, user_table_t, item_table_t,
                    u_out_t, i_out_t, idx_v, tiles_v, out_u, out_i,
                    sem, wb_sem):
    wid = lax.axis_index("s") * _NC + lax.axis_index("c")
    base = wid * _BPW
    iv = lax.iota(jnp.int32, 16)
    ngroups = _BPW // _G

    pltpu.sync_copy(user_ids.at[pl.ds(base, _BPW)],
                    idx_v.at[pl.ds(0, _BPW)])
    pltpu.sync_copy(item_ids.at[pl.ds(base, _BPW)],
                    idx_v.at[pl.ds(_BPW, _BPW)])

    def fire(tbl_t, v16, j):
        start = pl.multiple_of(
            lax.shift_right_logical(v16[j], 7) * 128, 128)
        pltpu.async_copy(tbl_t.at[:, pl.ds(start, 128)],
                         tiles_v.at[j], sem)

    def drain(tbl_t, out_v, v16, g, j):
        pltpu.make_async_copy(tbl_t.at[:, pl.ds(0, 128)],
                              tiles_v.at[j], sem).wait()
        slot16 = jnp.full((16,), j, jnp.int32)
        lane16 = jnp.full((16,), v16[j] & 127, jnp.int32)
        col16 = jnp.full((16,), g * _G + j, jnp.int32)
        for h in range(2):
            rows16 = iv + h * 16
            v = plsc.load_gather(tiles_v, [slot16, rows16, lane16])
            plsc.store_scatter(out_v, [rows16, col16], v)

    def loop_body(tbl_t, out_v, off):
        def body(g, v_prev):
            v16 = idx_v[pl.ds(off + g * _G, _G)]
            for j in range(_G):
                drain(tbl_t, out_v, v_prev, g - 1, j)
                fire(tbl_t, v16, j)
            return v16
        return body

    vu0 = idx_v[pl.ds(0, _G)]
    for j in range(_G):
        fire(user_table_t, vu0, j)
    vu_last = lax.fori_loop(1, ngroups, loop_body(user_table_t, out_u, 0),
                            vu0)
    vi0 = idx_v[pl.ds(_BPW, _G)]
    for j in range(_G):
        drain(user_table_t, out_u, vu_last, ngroups - 1, j)
        fire(item_table_t, vi0, j)
    cp_u = pltpu.async_copy(out_u, u_out_t.at[:, pl.ds(base, _BPW)], wb_sem)
    vi_last = lax.fori_loop(1, ngroups,
                            loop_body(item_table_t, out_i, _BPW), vi0)
    for j in range(_G):
        drain(item_table_t, out_i, vi_last, ngroups - 1, j)
    pltpu.sync_copy(out_i, i_out_t.at[:, pl.ds(base, _BPW)])
    cp_u.wait()


def _mm_body(u_ref, it_ref, o_ref):
    o_ref[...] = lax.dot_general(
        u_ref[...], it_ref[...], (((0,), (0,)), ((), ())),
        preferred_element_type=jnp.float32)


_BM = 1024
_BN = 4096

_matmul = pl.pallas_call(
    _mm_body,
    grid=(_B // _BM,),
    in_specs=[
        pl.BlockSpec((_D, _BM), lambda i: (0, i)),
        pl.BlockSpec((_D, _BN), lambda i: (0, 0)),
    ],
    out_specs=pl.BlockSpec((_BM, _BN), lambda i: (i, 0)),
    out_shape=jax.ShapeDtypeStruct((_B, _B), jnp.float32),
)


def kernel(user_ids, item_ids, user_table, item_table):
    u_lat_t, i_lat_t = _gather_latents(user_ids, item_ids,
                                       user_table.T, item_table.T)
    return _matmul(u_lat_t, i_lat_t)
